# Initial kernel scaffold; baseline (speedup 1.0000x reference)
import jax
import jax.numpy as jnp
from jax import lax
from jax.experimental import pallas as pl
from jax.experimental.pallas import tpu as pltpu
from jax.experimental.pallas import tpu_sc as plsc

N = 1048576
RES = (128, 256, 512, 1024)
NF = 4
NC, NS = 2, 16
NW = NC * NS
PW = N // NW
CHUNK = 128
NCHUNKS = PW // CHUNK
NG = CHUNK // 16


def _sc_body(x_hbm, t0, t1, t2, t3, out_hbm, xbuf, idxbuf, wbuf, gbuf,
             outbuf, sem):
    tables = (t0, t1, t2, t3)
    wid = lax.axis_index("s") * NC + lax.axis_index("c")
    iota = lax.iota(jnp.int32, 16)

    def chunk_body(c, carry):
        base = (wid * NCHUNKS + c) * CHUNK
        pltpu.sync_copy(x_hbm.at[pl.ds(base * 2, CHUNK * 2)], xbuf)

        def idx_body(g, carry2):
            p2 = (g * 16 + iota) * 2
            x0 = plsc.load_gather(xbuf, [p2])
            x1 = plsc.load_gather(xbuf, [p2 + 1])
            for o, r in enumerate(RES):
                rf = float(r)
                ii = []
                for d, xd in ((0, x0), (1, x1)):
                    xr = xd * rf
                    xr = jnp.where(xr >= rf, xr - rf, xr)
                    iv = xr.astype(jnp.int32)
                    fr = xr - iv.astype(jnp.float32)
                    w = (3.0 - 2.0 * fr) * fr * fr
                    wbuf[o * 2 + d, pl.ds(g * 16, 16)] = w
                    ii.append(iv)
                bidx = ii[0] * (r + 1) + ii[1]
                for ci, off in enumerate((0, 1, r + 1, r + 2)):
                    idxbuf[o * 4 + ci, pl.ds(g * 16, 16)] = bidx + off
            return carry2

        lax.fori_loop(0, NG, idx_body, 0)

        copies = []
        for o in range(4):
            for ci in range(4):
                k = o * 4 + ci
                copies.append(
                    pltpu.async_copy(tables[o].at[idxbuf.at[k]], gbuf.at[k],
                                     sem))
        for cp in copies:
            cp.wait()

        def lerp_body(g, carry2):
            p = g * 16 + iota
            accs = [None] * NF
            for o in range(4):
                w0 = wbuf[o * 2 + 0, pl.ds(g * 16, 16)]
                w1 = wbuf[o * 2 + 1, pl.ds(g * 16, 16)]
                amp = 128.0 / RES[o]
                for f in range(NF):
                    fidx = jnp.full((16,), f, jnp.int32)
                    v00 = plsc.load_gather(gbuf.at[o * 4 + 0], [p, fidx])
                    v01 = plsc.load_gather(gbuf.at[o * 4 + 1], [p, fidx])
                    v10 = plsc.load_gather(gbuf.at[o * 4 + 2], [p, fidx])
                    v11 = plsc.load_gather(gbuf.at[o * 4 + 3], [p, fidx])
                    va = v00 + w0 * (v10 - v00)
                    vb = v01 + w0 * (v11 - v01)
                    rv = (va + w1 * (vb - va)) * amp
                    accs[f] = rv if accs[f] is None else accs[f] + rv
            for f in range(NF):
                plsc.store_scatter(outbuf, [p * 4 + f], accs[f])
            return carry2

        lax.fori_loop(0, NG, lerp_body, 0)

        pltpu.sync_copy(outbuf, out_hbm.at[pl.ds(base * 4, CHUNK * 4)])
        return carry

    lax.fori_loop(0, NCHUNKS, chunk_body, 0)


def kernel(x, values_0, values_1, values_2, values_3):
    mesh = plsc.VectorSubcoreMesh(core_axis_name="c", subcore_axis_name="s")
    run = pl.kernel(
        _sc_body,
        out_type=jax.ShapeDtypeStruct((N * NF,), jnp.float32),
        mesh=mesh,
        scratch_types=[
            pltpu.VMEM((CHUNK * 2,), jnp.float32),
            pltpu.VMEM((16, CHUNK), jnp.int32),
            pltpu.VMEM((8, CHUNK), jnp.float32),
            pltpu.VMEM((16, CHUNK, NF), jnp.float32),
            pltpu.VMEM((CHUNK * NF,), jnp.float32),
            pltpu.SemaphoreType.DMA,
        ],
    )
    out = run(
        x.reshape(-1),
        values_0.reshape(-1, NF),
        values_1.reshape(-1, NF),
        values_2.reshape(-1, NF),
        values_3.reshape(-1, NF),
    )
    return out.reshape(N, NF)


# speedup vs baseline: 68.3095x; 68.3095x over previous
import jax
import jax.numpy as jnp
from jax import lax
from jax.experimental import pallas as pl
from jax.experimental.pallas import tpu as pltpu
from jax.experimental.pallas import tpu_sc as plsc

N = 1048576
RES = (128, 256, 512, 1024)
NF = 4
NC, NS = 2, 16
NW = NC * NS
PW = N // NW
CHUNK = 128
NCHUNKS = PW // CHUNK
NG = CHUNK // 16
BLOCK_I = {128: 4, 256: 8, 512: 8, 1024: 4}

_CPARAMS = pltpu.CompilerParams(needs_layout_passes=False,
                                use_tc_tiling_on_sc=False)


def _repack_body(v0, v1, v2, v3, c0, c1, c2, c3, sem):
    values = (v0, v1, v2, v3)
    cells = (c0, c1, c2, c3)
    wid = lax.axis_index("s") * NC + lax.axis_index("c")

    for o, r in enumerate(RES):
        rows_per = r // NW
        bi = BLOCK_I[r]

        def scoped(buf, o=o, r=r, rows_per=rows_per, bi=bi):
            def blk_body(b, carry):
                i0 = wid * rows_per + b * bi
                for c in range(4):
                    pltpu.sync_copy(
                        values[o].at[pl.ds(i0 + (c >> 1), bi),
                                     pl.ds(c & 1, r), :],
                        buf.at[:, :, pl.ds(c * NF, NF)])
                pltpu.sync_copy(buf, cells[o].at[pl.ds(i0, bi), :, :])
                return carry

            lax.fori_loop(0, rows_per // bi, blk_body, 0)

        pl.run_scoped(scoped, pltpu.VMEM((bi, r, 4 * NF), jnp.float32))


def _main_body(xt_hbm, c0, c1, c2, c3, out_hbm, x0buf, x1buf, idxbuf, wbuf,
               gbuf, outbuf, sem):
    cells = (c0, c1, c2, c3)
    wid = lax.axis_index("s") * NC + lax.axis_index("c")
    iota = lax.iota(jnp.int32, 16)

    def chunk_body(c, carry):
        base = (wid * NCHUNKS + c) * CHUNK
        pltpu.sync_copy(xt_hbm.at[0, pl.ds(base, CHUNK)], x0buf)
        pltpu.sync_copy(xt_hbm.at[1, pl.ds(base, CHUNK)], x1buf)

        def idx_body(g, carry2):
            x0 = x0buf[pl.ds(g * 16, 16)]
            x1 = x1buf[pl.ds(g * 16, 16)]
            for o, r in enumerate(RES):
                rf = float(r)
                ii = []
                for d, xd in ((0, x0), (1, x1)):
                    xr = xd * rf
                    xr = jnp.where(xr >= rf, xr - rf, xr)
                    iv = xr.astype(jnp.int32)
                    fr = xr - iv.astype(jnp.float32)
                    w = (3.0 - 2.0 * fr) * fr * fr
                    wbuf[o * 2 + d, pl.ds(g * 16, 16)] = w
                    ii.append(iv)
                idxbuf[o, pl.ds(g * 16, 16)] = ii[0] * r + ii[1]
            return carry2

        lax.fori_loop(0, NG, idx_body, 0)

        copies = [
            pltpu.async_copy(cells[o].at[idxbuf.at[o]], gbuf.at[o], sem)
            for o in range(4)
        ]
        for cp in copies:
            cp.wait()

        def lerp_body(g, carry2):
            p = g * 16 + iota
            accs = [None] * NF
            for o in range(4):
                w0 = wbuf[o * 2 + 0, pl.ds(g * 16, 16)]
                w1 = wbuf[o * 2 + 1, pl.ds(g * 16, 16)]
                amp = 128.0 / RES[o]
                for f in range(NF):
                    v00 = plsc.load_gather(gbuf.at[o], [p, jnp.full((16,), f, jnp.int32)])
                    v01 = plsc.load_gather(gbuf.at[o], [p, jnp.full((16,), 4 + f, jnp.int32)])
                    v10 = plsc.load_gather(gbuf.at[o], [p, jnp.full((16,), 8 + f, jnp.int32)])
                    v11 = plsc.load_gather(gbuf.at[o], [p, jnp.full((16,), 12 + f, jnp.int32)])
                    va = v00 + w0 * (v10 - v00)
                    vb = v01 + w0 * (v11 - v01)
                    rv = (va + w1 * (vb - va)) * amp
                    accs[f] = rv if accs[f] is None else accs[f] + rv
            for f in range(NF):
                plsc.store_scatter(outbuf, [p, jnp.full((16,), f, jnp.int32)],
                                   accs[f])
            return carry2

        lax.fori_loop(0, NG, lerp_body, 0)

        pltpu.sync_copy(outbuf, out_hbm.at[pl.ds(base, CHUNK), :])
        return carry

    lax.fori_loop(0, NCHUNKS, chunk_body, 0)


def kernel(x, values_0, values_1, values_2, values_3):
    mesh = plsc.VectorSubcoreMesh(core_axis_name="c", subcore_axis_name="s")

    repack = pl.kernel(
        _repack_body,
        out_type=tuple(
            jax.ShapeDtypeStruct((r, r, 4 * NF), jnp.float32) for r in RES),
        mesh=mesh,
        scratch_types=[pltpu.SemaphoreType.DMA],
        compiler_params=_CPARAMS,
    )
    cells = repack(values_0, values_1, values_2, values_3)

    main = pl.kernel(
        _main_body,
        out_type=jax.ShapeDtypeStruct((N, NF), jnp.float32),
        mesh=mesh,
        scratch_types=[
            pltpu.VMEM((CHUNK,), jnp.float32),
            pltpu.VMEM((CHUNK,), jnp.float32),
            pltpu.VMEM((4, CHUNK), jnp.int32),
            pltpu.VMEM((8, CHUNK), jnp.float32),
            pltpu.VMEM((4, CHUNK, 4 * NF), jnp.float32),
            pltpu.VMEM((CHUNK, NF), jnp.float32),
            pltpu.SemaphoreType.DMA,
        ],
        compiler_params=_CPARAMS,
    )
    return main(x.T, *(cl.reshape(-1, 4 * NF) for cl in cells))
